# baseline (speedup 1.0000x reference)
import functools

import jax
import jax.numpy as jnp
from jax import lax
from jax.experimental import pallas as pl
from jax.experimental.pallas import tpu as pltpu
from jax.experimental.pallas import tpu_sc as plsc

N = 10000
E = 320000
D = 128

NC = 2
NS = 16
NW = NC * NS
E_PER_W = E // NW
CHUNK = 80
N_CHUNKS = E_PER_W // CHUNK
NPAD = 10240
ROWS_PER_TILE = NPAD // NS


def _sc_segsum_kernel(y_hbm, src_hbm, dst_hbm, z2_hbm, z1_hbm,
                      *refs, with_deg):
    if with_deg:
        (s_out, deg_out, src_v, dst_v, rows_v, ones_v, acc_sp, deg_sp,
         si, sg, ss) = refs
    else:
        (s_out, src_v, dst_v, rows_v, ones_v, acc_sp, si, sg, ss) = refs
        deg_out = deg_sp = None
    cid = lax.axis_index("c")
    sid = lax.axis_index("s")
    wid = cid * NS + sid
    n_desc = 2 if with_deg else 1

    row0 = sid * ROWS_PER_TILE
    pltpu.sync_copy(z2_hbm, acc_sp.at[pl.ds(row0, ROWS_PER_TILE)])
    if with_deg:
        pltpu.sync_copy(z1_hbm, deg_sp.at[pl.ds(row0, ROWS_PER_TILE)])
        for i in range(CHUNK // 16):
            ones_v[pl.ds(i * 16, 16)] = jnp.ones((16,), jnp.float32)

    plsc.subcore_barrier()

    def issue_idx(k, b):
        base = wid * E_PER_W + k * CHUNK
        pltpu.async_copy(src_hbm.at[pl.ds(base, CHUNK)], src_v.at[b], si.at[b])
        pltpu.async_copy(dst_hbm.at[pl.ds(base, CHUNK)], dst_v.at[b], si.at[b])

    def wait_idx(b):
        pltpu.make_async_copy(src_hbm.at[pl.ds(0, CHUNK)], src_v.at[b],
                              si.at[b]).wait()
        pltpu.make_async_copy(dst_hbm.at[pl.ds(0, CHUNK)], dst_v.at[b],
                              si.at[b]).wait()

    def issue_gather(b):
        pltpu.async_copy(y_hbm.at[src_v.at[b]], rows_v.at[b], sg.at[b])

    def wait_gather(b):
        pltpu.make_async_copy(y_hbm.at[src_v.at[b]], rows_v.at[b],
                              sg.at[b]).wait()

    def issue_scatter(b):
        pltpu.async_copy(rows_v.at[b], acc_sp.at[dst_v.at[b]], ss.at[b],
                         add=True)
        if with_deg:
            pltpu.async_copy(ones_v, deg_sp.at[dst_v.at[b]], ss.at[b], add=True)

    def drain_scatter(b):
        pltpu.make_async_copy(rows_v.at[b], acc_sp.at[dst_v.at[b]],
                              ss.at[b]).wait()
        if with_deg:
            pltpu.make_async_copy(ones_v, deg_sp.at[dst_v.at[b]],
                                  ss.at[b]).wait()

    def stage(k, *, drain):
        b, bm1, bm2 = k % 3, (k - 1) % 3, (k - 2) % 3
        wait_gather(bm2)
        issue_scatter(bm2)
        if drain:
            drain_scatter(b)
        issue_idx(k, b)
        wait_idx(bm1)
        issue_gather(bm1)

    issue_idx(0, 0)
    issue_idx(1, 1)
    wait_idx(0)
    issue_gather(0)
    stage(2, drain=False)
    stage(3, drain=True)
    stage(4, drain=True)

    def body(j, _):
        k = 5 + 3 * j
        stage(k, drain=True)
        stage(k + 1, drain=True)
        stage(k + 2, drain=True)
        return _

    lax.fori_loop(0, (N_CHUNKS - 5) // 3, body, None)

    kl = N_CHUNKS - 1
    wait_gather((kl - 1) % 3)
    issue_scatter((kl - 1) % 3)
    wait_idx(kl % 3)
    issue_gather(kl % 3)
    wait_gather(kl % 3)
    issue_scatter(kl % 3)
    drain_scatter(0)
    drain_scatter(1)
    drain_scatter(2)

    plsc.subcore_barrier()

    pltpu.sync_copy(acc_sp.at[pl.ds(row0, ROWS_PER_TILE)],
                    s_out.at[cid, pl.ds(row0, ROWS_PER_TILE)])
    if with_deg:
        pltpu.sync_copy(deg_sp.at[pl.ds(row0, ROWS_PER_TILE)],
                        deg_out.at[cid, pl.ds(row0, ROWS_PER_TILE)])


assert (N_CHUNKS - 5) % 3 == 0


def _sc_segsum(y, src, dst, with_deg):
    z2 = jnp.zeros((ROWS_PER_TILE, D), jnp.float32)
    z1 = jnp.zeros((ROWS_PER_TILE,), jnp.float32)
    mesh = plsc.VectorSubcoreMesh(core_axis_name="c", subcore_axis_name="s",
                                  num_cores=NC, num_subcores=NS)
    out_type = [jax.ShapeDtypeStruct((NC, NPAD, D), jnp.float32)]
    scratch = [
        pltpu.VMEM((3, CHUNK), jnp.int32),
        pltpu.VMEM((3, CHUNK), jnp.int32),
        pltpu.VMEM((3, CHUNK, D), jnp.float32),
        pltpu.VMEM((CHUNK,), jnp.float32),
        pltpu.VMEM_SHARED((NPAD, D), jnp.float32),
        pltpu.SemaphoreType.DMA((3,)),
        pltpu.SemaphoreType.DMA((3,)),
        pltpu.SemaphoreType.DMA((3,)),
    ]
    if with_deg:
        out_type.append(jax.ShapeDtypeStruct((NC, NPAD), jnp.float32))
        scratch.insert(5, pltpu.VMEM_SHARED((NPAD,), jnp.float32))
    fn = pl.kernel(
        functools.partial(_sc_segsum_kernel, with_deg=with_deg),
        out_type=out_type,
        mesh=mesh,
        scratch_types=scratch,
    )
    return fn(y, src, dst, z2, z1)


ROW_BLK = 2000
N_BLKS = N // ROW_BLK


def _tc_right_kernel(x_ref, w_ref, b_ref, o_ref):
    o_ref[...] = jnp.dot(x_ref[...], w_ref[...],
                         preferred_element_type=jnp.float32) + b_ref[...]


def _tc_right(x, w, b):
    return pl.pallas_call(
        _tc_right_kernel,
        grid=(N_BLKS,),
        in_specs=[
            pl.BlockSpec((ROW_BLK, D), lambda i: (i, 0)),
            pl.BlockSpec((D, D), lambda i: (0, 0)),
            pl.BlockSpec((1, D), lambda i: (0, 0)),
        ],
        out_specs=pl.BlockSpec((ROW_BLK, D), lambda i: (i, 0)),
        out_shape=jax.ShapeDtypeStruct((N, D), jnp.float32),
    )(x, w, b)


def _tc_layer_kernel(sp_ref, dp_ref, xw_ref, w_ref,
                     h_ref, stats_ref, *, with_stats):
    s = sp_ref[0] + sp_ref[1]
    deg = dp_ref[0] + dp_ref[1]
    recip = 1.0 / jnp.maximum(deg, 1.0)
    agg = s * recip
    h = jnp.dot(agg, w_ref[...],
                preferred_element_type=jnp.float32) + xw_ref[...]
    h_ref[...] = h
    if with_stats:
        i = pl.program_id(0)

        @pl.when(i == 0)
        def _():
            stats_ref[...] = jnp.zeros_like(stats_ref)

        stats_ref[0:1, :] += jnp.sum(h, axis=0, keepdims=True)
        stats_ref[1:2, :] += jnp.sum(h * h, axis=0, keepdims=True)


def _tc_layer(s_part, deg_part, xw, w_l, with_stats):
    dp = deg_part.reshape(NC, NPAD, 1)
    out_shape = [jax.ShapeDtypeStruct((N, D), jnp.float32)]
    out_specs = [pl.BlockSpec((ROW_BLK, D), lambda i: (i, 0))]
    if with_stats:
        out_shape.append(jax.ShapeDtypeStruct((2, D), jnp.float32))
        out_specs.append(pl.BlockSpec((2, D), lambda i: (0, 0)))
    kfn = functools.partial(_tc_layer_kernel, with_stats=with_stats)
    if not with_stats:
        def kfn(sp, dp_, xw_, w_, h_):
            _tc_layer_kernel(sp, dp_, xw_, w_, h_, None, with_stats=False)
    res = pl.pallas_call(
        kfn,
        grid=(N_BLKS,),
        in_specs=[
            pl.BlockSpec((NC, ROW_BLK, D), lambda i: (0, i, 0)),
            pl.BlockSpec((NC, ROW_BLK, 1), lambda i: (0, i, 0)),
            pl.BlockSpec((ROW_BLK, D), lambda i: (i, 0)),
            pl.BlockSpec((D, D), lambda i: (0, 0)),
        ],
        out_specs=out_specs if with_stats else out_specs[0],
        out_shape=out_shape if with_stats else out_shape[0],
    )(s_part, dp, xw, w_l)
    return res


def _tc_bn_relu_kernel(h_ref, stats_ref, g_ref, bt_ref, o_ref):
    mean = stats_ref[0:1, :] / N
    var = stats_ref[1:2, :] / N - mean * mean
    rstd = lax.rsqrt(var + 1e-5)
    o_ref[...] = jnp.maximum(
        (h_ref[...] - mean) * rstd * g_ref[...] + bt_ref[...], 0.0)


def _tc_bn_relu(h_pre, stats, gamma, beta):
    return pl.pallas_call(
        _tc_bn_relu_kernel,
        grid=(N_BLKS,),
        in_specs=[
            pl.BlockSpec((ROW_BLK, D), lambda i: (i, 0)),
            pl.BlockSpec((2, D), lambda i: (0, 0)),
            pl.BlockSpec((1, D), lambda i: (0, 0)),
            pl.BlockSpec((1, D), lambda i: (0, 0)),
        ],
        out_specs=pl.BlockSpec((ROW_BLK, D), lambda i: (i, 0)),
        out_shape=jax.ShapeDtypeStruct((N, D), jnp.float32),
    )(h_pre, stats, gamma, beta)


def kernel(x, edge_index, W1_l, b1_l, W1_r, gamma, beta, W2_l, b2_l, W2_r):
    src = edge_index[0]
    dst = edge_index[1]
    b1 = b1_l.reshape(1, D)
    b2 = b2_l.reshape(1, D)
    g2 = gamma.reshape(1, D)
    bt2 = beta.reshape(1, D)

    s1, deg = _sc_segsum(x, src, dst, with_deg=True)
    xw1 = _tc_right(x, W1_r, b1)
    h_pre, stats = _tc_layer(s1, deg, xw1, W1_l, with_stats=True)
    h = _tc_bn_relu(h_pre, stats, g2, bt2)
    (s2,) = _sc_segsum(h, src, dst, with_deg=False)
    hw2 = _tc_right(h, W2_r, b2)
    out = _tc_layer(s2, deg, hw2, W2_l, with_stats=False)
    return out


# speedup vs baseline: 1.0183x; 1.0183x over previous
import functools

import jax
import jax.numpy as jnp
from jax import lax
from jax.experimental import pallas as pl
from jax.experimental.pallas import tpu as pltpu
from jax.experimental.pallas import tpu_sc as plsc

N = 10000
E = 320000
D = 128

NC = 2
NS = 16
NW = NC * NS
E_PER_W = E // NW
CHUNK = 80
N_CHUNKS = E_PER_W // CHUNK
NPAD = 10240
ROWS_PER_TILE = NPAD // NS


def _sc_segsum_kernel(y_hbm, src_hbm, dst_hbm, z2_hbm, z1_hbm,
                      *refs, with_deg):
    if with_deg:
        (s_out, deg_out, src_v, dst_v, rows_v, ones_v, acc_sp, deg_sp,
         si, sg, ss) = refs
    else:
        (s_out, src_v, dst_v, rows_v, ones_v, acc_sp, si, sg, ss) = refs
        deg_out = deg_sp = None
    cid = lax.axis_index("c")
    sid = lax.axis_index("s")
    wid = cid * NS + sid
    n_desc = 2 if with_deg else 1

    row0 = sid * ROWS_PER_TILE

    def issue_idx(k, b):
        base = wid * E_PER_W + k * CHUNK
        pltpu.async_copy(src_hbm.at[pl.ds(base, CHUNK)], src_v.at[b], si.at[b])
        pltpu.async_copy(dst_hbm.at[pl.ds(base, CHUNK)], dst_v.at[b], si.at[b])

    def wait_idx(b):
        pltpu.make_async_copy(src_hbm.at[pl.ds(0, CHUNK)], src_v.at[b],
                              si.at[b]).wait()
        pltpu.make_async_copy(dst_hbm.at[pl.ds(0, CHUNK)], dst_v.at[b],
                              si.at[b]).wait()

    def issue_gather(b):
        pltpu.async_copy(y_hbm.at[src_v.at[b]], rows_v.at[b], sg.at[b])

    def wait_gather(b):
        pltpu.make_async_copy(y_hbm.at[src_v.at[b]], rows_v.at[b],
                              sg.at[b]).wait()

    def issue_scatter(b):
        pltpu.async_copy(rows_v.at[b], acc_sp.at[dst_v.at[b]], ss.at[b],
                         add=True)
        if with_deg:
            pltpu.async_copy(ones_v, deg_sp.at[dst_v.at[b]], ss.at[b], add=True)

    def drain_scatter(b):
        pltpu.make_async_copy(rows_v.at[b], acc_sp.at[dst_v.at[b]],
                              ss.at[b]).wait()
        if with_deg:
            pltpu.make_async_copy(ones_v, deg_sp.at[dst_v.at[b]],
                                  ss.at[b]).wait()

    def stage(k, *, drain):
        b, bm1, bm2 = k % 3, (k - 1) % 3, (k - 2) % 3
        wait_gather(bm2)
        issue_scatter(bm2)
        if drain:
            drain_scatter(b)
        issue_idx(k, b)
        wait_idx(bm1)
        issue_gather(bm1)

    issue_idx(0, 0)
    issue_idx(1, 1)
    wait_idx(0)
    issue_gather(0)

    pltpu.sync_copy(z2_hbm, acc_sp.at[pl.ds(row0, ROWS_PER_TILE)])
    if with_deg:
        pltpu.sync_copy(z1_hbm, deg_sp.at[pl.ds(row0, ROWS_PER_TILE)])
        for i in range(CHUNK // 16):
            ones_v[pl.ds(i * 16, 16)] = jnp.ones((16,), jnp.float32)

    plsc.subcore_barrier()

    stage(2, drain=False)
    stage(3, drain=True)
    stage(4, drain=True)

    def body(j, _):
        k = 5 + 3 * j
        stage(k, drain=True)
        stage(k + 1, drain=True)
        stage(k + 2, drain=True)
        return _

    lax.fori_loop(0, (N_CHUNKS - 5) // 3, body, None)

    kl = N_CHUNKS - 1
    wait_gather((kl - 1) % 3)
    issue_scatter((kl - 1) % 3)
    wait_idx(kl % 3)
    issue_gather(kl % 3)
    wait_gather(kl % 3)
    issue_scatter(kl % 3)
    drain_scatter(0)
    drain_scatter(1)
    drain_scatter(2)

    plsc.subcore_barrier()

    pltpu.sync_copy(acc_sp.at[pl.ds(row0, ROWS_PER_TILE)],
                    s_out.at[cid, pl.ds(row0, ROWS_PER_TILE)])
    if with_deg:
        pltpu.sync_copy(deg_sp.at[pl.ds(row0, ROWS_PER_TILE)],
                        deg_out.at[cid, pl.ds(row0, ROWS_PER_TILE)])


assert (N_CHUNKS - 5) % 3 == 0


def _sc_segsum(y, src, dst, with_deg):
    z2 = jnp.zeros((ROWS_PER_TILE, D), jnp.float32)
    z1 = jnp.zeros((ROWS_PER_TILE,), jnp.float32)
    mesh = plsc.VectorSubcoreMesh(core_axis_name="c", subcore_axis_name="s",
                                  num_cores=NC, num_subcores=NS)
    out_type = [jax.ShapeDtypeStruct((NC, NPAD, D), jnp.float32)]
    scratch = [
        pltpu.VMEM((3, CHUNK), jnp.int32),
        pltpu.VMEM((3, CHUNK), jnp.int32),
        pltpu.VMEM((3, CHUNK, D), jnp.float32),
        pltpu.VMEM((CHUNK,), jnp.float32),
        pltpu.VMEM_SHARED((NPAD, D), jnp.float32),
        pltpu.SemaphoreType.DMA((3,)),
        pltpu.SemaphoreType.DMA((3,)),
        pltpu.SemaphoreType.DMA((3,)),
    ]
    if with_deg:
        out_type.append(jax.ShapeDtypeStruct((NC, NPAD), jnp.float32))
        scratch.insert(5, pltpu.VMEM_SHARED((NPAD,), jnp.float32))
    fn = pl.kernel(
        functools.partial(_sc_segsum_kernel, with_deg=with_deg),
        out_type=out_type,
        mesh=mesh,
        scratch_types=scratch,
    )
    return fn(y, src, dst, z2, z1)


ROW_BLK = 2000
N_BLKS = N // ROW_BLK


def _tc_layer_kernel(sp_ref, dp_ref, x_ref, w_ref, b_ref,
                     h_ref, stats_ref, cat_ref, *, with_stats):
    s = sp_ref[0] + sp_ref[1]
    deg = dp_ref[0] + dp_ref[1]
    recip = 1.0 / jnp.maximum(deg, 1.0)
    cat_ref[:, :D] = s * recip
    cat_ref[:, D:] = x_ref[...]
    h = jnp.dot(cat_ref[...], w_ref[...],
                preferred_element_type=jnp.float32) + b_ref[...]
    h_ref[...] = h
    if with_stats:
        i = pl.program_id(0)

        @pl.when(i == 0)
        def _():
            stats_ref[...] = jnp.zeros_like(stats_ref)

        stats_ref[0:1, :] += jnp.sum(h, axis=0, keepdims=True)
        stats_ref[1:2, :] += jnp.sum(h * h, axis=0, keepdims=True)


def _tc_layer(s_part, deg_part, x, w_cat, b, with_stats):
    dp = deg_part.reshape(NC, NPAD, 1)
    out_shape = [jax.ShapeDtypeStruct((N, D), jnp.float32)]
    out_specs = [pl.BlockSpec((ROW_BLK, D), lambda i: (i, 0))]
    if with_stats:
        out_shape.append(jax.ShapeDtypeStruct((2, D), jnp.float32))
        out_specs.append(pl.BlockSpec((2, D), lambda i: (0, 0)))
    kfn = functools.partial(_tc_layer_kernel, with_stats=with_stats)
    if not with_stats:
        def kfn(sp, dp_, x_, w_, b_, h_, cat_):
            _tc_layer_kernel(sp, dp_, x_, w_, b_, h_, None, cat_,
                             with_stats=False)
    res = pl.pallas_call(
        kfn,
        grid=(N_BLKS,),
        in_specs=[
            pl.BlockSpec((NC, ROW_BLK, D), lambda i: (0, i, 0)),
            pl.BlockSpec((NC, ROW_BLK, 1), lambda i: (0, i, 0)),
            pl.BlockSpec((ROW_BLK, D), lambda i: (i, 0)),
            pl.BlockSpec((2 * D, D), lambda i: (0, 0)),
            pl.BlockSpec((1, D), lambda i: (0, 0)),
        ],
        out_specs=out_specs if with_stats else out_specs[0],
        out_shape=out_shape if with_stats else out_shape[0],
        scratch_shapes=[pltpu.VMEM((ROW_BLK, 2 * D), jnp.float32)],
    )(s_part, dp, x, w_cat, b)
    return res


def _tc_bn_relu_kernel(h_ref, stats_ref, g_ref, bt_ref, o_ref):
    mean = stats_ref[0:1, :] / N
    var = stats_ref[1:2, :] / N - mean * mean
    rstd = lax.rsqrt(var + 1e-5)
    o_ref[...] = jnp.maximum(
        (h_ref[...] - mean) * rstd * g_ref[...] + bt_ref[...], 0.0)


def _tc_bn_relu(h_pre, stats, gamma, beta):
    return pl.pallas_call(
        _tc_bn_relu_kernel,
        grid=(N_BLKS,),
        in_specs=[
            pl.BlockSpec((ROW_BLK, D), lambda i: (i, 0)),
            pl.BlockSpec((2, D), lambda i: (0, 0)),
            pl.BlockSpec((1, D), lambda i: (0, 0)),
            pl.BlockSpec((1, D), lambda i: (0, 0)),
        ],
        out_specs=pl.BlockSpec((ROW_BLK, D), lambda i: (i, 0)),
        out_shape=jax.ShapeDtypeStruct((N, D), jnp.float32),
    )(h_pre, stats, gamma, beta)


def kernel(x, edge_index, W1_l, b1_l, W1_r, gamma, beta, W2_l, b2_l, W2_r):
    src = edge_index[0]
    dst = edge_index[1]
    w1 = jnp.concatenate([W1_l, W1_r], axis=0)
    w2 = jnp.concatenate([W2_l, W2_r], axis=0)
    b1 = b1_l.reshape(1, D)
    b2 = b2_l.reshape(1, D)
    g2 = gamma.reshape(1, D)
    bt2 = beta.reshape(1, D)

    s1, deg = _sc_segsum(x, src, dst, with_deg=True)
    h_pre, stats = _tc_layer(s1, deg, x, w1, b1, with_stats=True)
    h = _tc_bn_relu(h_pre, stats, g2, bt2)
    (s2,) = _sc_segsum(h, src, dst, with_deg=False)
    out = _tc_layer(s2, deg, h, w2, b2, with_stats=False)
    return out


# speedup vs baseline: 1.1633x; 1.1424x over previous
import functools

import jax
import jax.numpy as jnp
from jax import lax
from jax.experimental import pallas as pl
from jax.experimental.pallas import tpu as pltpu
from jax.experimental.pallas import tpu_sc as plsc

N = 10000
E = 320000
D = 128

NC = 2
NS = 16
NW = NC * NS
E_PER_W = E // NW
CHUNK = 120
N_CHUNKS = E_PER_W // CHUNK
TAIL = E_PER_W - N_CHUNKS * CHUNK
NPAD = 10240
ROWS_PER_TILE = NPAD // NS


def _sc_segsum_kernel(y_hbm, src_hbm, dst_hbm, z2_hbm, z1_hbm,
                      *refs, with_deg):
    if with_deg:
        (s_out, deg_out, src_v, dst_v, rows_v, ones_v,
         src_t, dst_t, acc_sp, deg_sp, si, sg, ss) = refs
    else:
        (s_out, src_v, dst_v, rows_v, ones_v,
         src_t, dst_t, acc_sp, si, sg, ss) = refs
        deg_out = deg_sp = None
    cid = lax.axis_index("c")
    sid = lax.axis_index("s")
    wid = cid * NS + sid
    n_desc = 2 if with_deg else 1

    row0 = sid * ROWS_PER_TILE

    def issue_idx(k, b):
        base = wid * E_PER_W + k * CHUNK
        pltpu.async_copy(src_hbm.at[pl.ds(base, CHUNK)], src_v.at[b], si.at[b])
        pltpu.async_copy(dst_hbm.at[pl.ds(base, CHUNK)], dst_v.at[b], si.at[b])

    def wait_idx(b):
        pltpu.make_async_copy(src_hbm.at[pl.ds(0, CHUNK)], src_v.at[b],
                              si.at[b]).wait()
        pltpu.make_async_copy(dst_hbm.at[pl.ds(0, CHUNK)], dst_v.at[b],
                              si.at[b]).wait()

    def issue_gather(b):
        pltpu.async_copy(y_hbm.at[src_v.at[b]], rows_v.at[b], sg.at[b])

    def wait_gather(b):
        pltpu.make_async_copy(y_hbm.at[src_v.at[b]], rows_v.at[b],
                              sg.at[b]).wait()

    def issue_scatter(b):
        pltpu.async_copy(rows_v.at[b], acc_sp.at[dst_v.at[b]], ss.at[b],
                         add=True)
        if with_deg:
            pltpu.async_copy(ones_v, deg_sp.at[dst_v.at[b]], ss.at[b], add=True)

    def drain_scatter(b):
        pltpu.make_async_copy(rows_v.at[b], acc_sp.at[dst_v.at[b]],
                              ss.at[b]).wait()
        if with_deg:
            pltpu.make_async_copy(ones_v, deg_sp.at[dst_v.at[b]],
                                  ss.at[b]).wait()

    def stage(k, *, drain):
        b, bm1, bm2 = k % 3, (k - 1) % 3, (k - 2) % 3
        wait_gather(bm2)
        issue_scatter(bm2)
        if drain:
            drain_scatter(b)
        issue_idx(k, b)
        wait_idx(bm1)
        issue_gather(bm1)

    issue_idx(0, 0)
    issue_idx(1, 1)
    wait_idx(0)
    issue_gather(0)

    pltpu.sync_copy(z2_hbm, acc_sp.at[pl.ds(row0, ROWS_PER_TILE)])
    if with_deg:
        pltpu.sync_copy(z1_hbm, deg_sp.at[pl.ds(row0, ROWS_PER_TILE)])
        for i in range(CHUNK // 16):
            ones_v[pl.ds(i * 16, 16)] = jnp.ones((16,), jnp.float32)

    plsc.subcore_barrier()

    stage(2, drain=False)
    stage(3, drain=True)
    stage(4, drain=True)

    def body(j, _):
        k = 5 + 3 * j
        stage(k, drain=True)
        stage(k + 1, drain=True)
        stage(k + 2, drain=True)
        return _

    n_loop = (N_CHUNKS - 5) // 3
    lax.fori_loop(0, n_loop, body, None)
    for k in range(5 + 3 * n_loop, N_CHUNKS):
        stage(k, drain=True)

    kl = N_CHUNKS - 1
    wait_gather((kl - 1) % 3)
    issue_scatter((kl - 1) % 3)
    wait_idx(kl % 3)
    issue_gather(kl % 3)
    wait_gather(kl % 3)
    issue_scatter(kl % 3)
    drain_scatter(0)
    drain_scatter(1)
    drain_scatter(2)

    tbase = wid * E_PER_W + N_CHUNKS * CHUNK
    pltpu.sync_copy(src_hbm.at[pl.ds(tbase, TAIL)], src_t)
    pltpu.sync_copy(dst_hbm.at[pl.ds(tbase, TAIL)], dst_t)
    rows_tail = rows_v.at[0, pl.ds(0, TAIL)]
    pltpu.async_copy(y_hbm.at[src_t], rows_tail, sg.at[0]).wait()
    pltpu.sync_copy(rows_tail, acc_sp.at[dst_t], add=True)
    if with_deg:
        pltpu.sync_copy(ones_v.at[pl.ds(0, TAIL)], deg_sp.at[dst_t],
                        add=True)

    plsc.subcore_barrier()

    pltpu.sync_copy(acc_sp.at[pl.ds(row0, ROWS_PER_TILE)],
                    s_out.at[cid, pl.ds(row0, ROWS_PER_TILE)])
    if with_deg:
        pltpu.sync_copy(deg_sp.at[pl.ds(row0, ROWS_PER_TILE)],
                        deg_out.at[cid, pl.ds(row0, ROWS_PER_TILE)])


def _sc_segsum(y, src, dst, with_deg):
    z2 = jnp.zeros((ROWS_PER_TILE, D), jnp.float32)
    z1 = jnp.zeros((ROWS_PER_TILE,), jnp.float32)
    mesh = plsc.VectorSubcoreMesh(core_axis_name="c", subcore_axis_name="s",
                                  num_cores=NC, num_subcores=NS)
    out_type = [jax.ShapeDtypeStruct((NC, NPAD, D), jnp.float32)]
    scratch = [
        pltpu.VMEM((3, CHUNK), jnp.int32),
        pltpu.VMEM((3, CHUNK), jnp.int32),
        pltpu.VMEM((3, CHUNK, D), jnp.float32),
        pltpu.VMEM((CHUNK,), jnp.float32),
        pltpu.VMEM((TAIL,), jnp.int32),
        pltpu.VMEM((TAIL,), jnp.int32),
        pltpu.VMEM_SHARED((NPAD, D), jnp.float32),
        pltpu.SemaphoreType.DMA((3,)),
        pltpu.SemaphoreType.DMA((3,)),
        pltpu.SemaphoreType.DMA((3,)),
    ]
    if with_deg:
        out_type.append(jax.ShapeDtypeStruct((NC, NPAD), jnp.float32))
        scratch.insert(7, pltpu.VMEM_SHARED((NPAD,), jnp.float32))
    fn = pl.kernel(
        functools.partial(_sc_segsum_kernel, with_deg=with_deg),
        out_type=out_type,
        mesh=mesh,
        scratch_types=scratch,
    )
    return fn(y, src, dst, z2, z1)


ROW_BLK = 2000
N_BLKS = N // ROW_BLK


def _tc_layer_kernel(sp_ref, dp_ref, x_ref, w_ref, b_ref,
                     h_ref, stats_ref, cat_ref, *, with_stats):
    s = sp_ref[0] + sp_ref[1]
    deg = dp_ref[0] + dp_ref[1]
    recip = 1.0 / jnp.maximum(deg, 1.0)
    cat_ref[:, :D] = s * recip
    cat_ref[:, D:] = x_ref[...]
    h = jnp.dot(cat_ref[...], w_ref[...],
                preferred_element_type=jnp.float32) + b_ref[...]
    h_ref[...] = h
    if with_stats:
        i = pl.program_id(0)

        @pl.when(i == 0)
        def _():
            stats_ref[...] = jnp.zeros_like(stats_ref)

        stats_ref[0:1, :] += jnp.sum(h, axis=0, keepdims=True)
        stats_ref[1:2, :] += jnp.sum(h * h, axis=0, keepdims=True)


def _tc_layer(s_part, deg_part, x, w_cat, b, with_stats):
    dp = deg_part.reshape(NC, NPAD, 1)
    out_shape = [jax.ShapeDtypeStruct((N, D), jnp.float32)]
    out_specs = [pl.BlockSpec((ROW_BLK, D), lambda i: (i, 0))]
    if with_stats:
        out_shape.append(jax.ShapeDtypeStruct((2, D), jnp.float32))
        out_specs.append(pl.BlockSpec((2, D), lambda i: (0, 0)))
    kfn = functools.partial(_tc_layer_kernel, with_stats=with_stats)
    if not with_stats:
        def kfn(sp, dp_, x_, w_, b_, h_, cat_):
            _tc_layer_kernel(sp, dp_, x_, w_, b_, h_, None, cat_,
                             with_stats=False)
    res = pl.pallas_call(
        kfn,
        grid=(N_BLKS,),
        in_specs=[
            pl.BlockSpec((NC, ROW_BLK, D), lambda i: (0, i, 0)),
            pl.BlockSpec((NC, ROW_BLK, 1), lambda i: (0, i, 0)),
            pl.BlockSpec((ROW_BLK, D), lambda i: (i, 0)),
            pl.BlockSpec((2 * D, D), lambda i: (0, 0)),
            pl.BlockSpec((1, D), lambda i: (0, 0)),
        ],
        out_specs=out_specs if with_stats else out_specs[0],
        out_shape=out_shape if with_stats else out_shape[0],
        scratch_shapes=[pltpu.VMEM((ROW_BLK, 2 * D), jnp.float32)],
    )(s_part, dp, x, w_cat, b)
    return res


def _tc_bn_relu_kernel(h_ref, stats_ref, g_ref, bt_ref, o_ref):
    mean = stats_ref[0:1, :] / N
    var = stats_ref[1:2, :] / N - mean * mean
    rstd = lax.rsqrt(var + 1e-5)
    o_ref[...] = jnp.maximum(
        (h_ref[...] - mean) * rstd * g_ref[...] + bt_ref[...], 0.0)


def _tc_bn_relu(h_pre, stats, gamma, beta):
    return pl.pallas_call(
        _tc_bn_relu_kernel,
        grid=(N_BLKS,),
        in_specs=[
            pl.BlockSpec((ROW_BLK, D), lambda i: (i, 0)),
            pl.BlockSpec((2, D), lambda i: (0, 0)),
            pl.BlockSpec((1, D), lambda i: (0, 0)),
            pl.BlockSpec((1, D), lambda i: (0, 0)),
        ],
        out_specs=pl.BlockSpec((ROW_BLK, D), lambda i: (i, 0)),
        out_shape=jax.ShapeDtypeStruct((N, D), jnp.float32),
    )(h_pre, stats, gamma, beta)


def kernel(x, edge_index, W1_l, b1_l, W1_r, gamma, beta, W2_l, b2_l, W2_r):
    src = edge_index[0]
    dst = edge_index[1]
    w1 = jnp.concatenate([W1_l, W1_r], axis=0)
    w2 = jnp.concatenate([W2_l, W2_r], axis=0)
    b1 = b1_l.reshape(1, D)
    b2 = b2_l.reshape(1, D)
    g2 = gamma.reshape(1, D)
    bt2 = beta.reshape(1, D)

    s1, deg = _sc_segsum(x, src, dst, with_deg=True)
    h_pre, stats = _tc_layer(s1, deg, x, w1, b1, with_stats=True)
    h = _tc_bn_relu(h_pre, stats, g2, bt2)
    (s2,) = _sc_segsum(h, src, dst, with_deg=False)
    out = _tc_layer(s2, deg, h, w2, b2, with_stats=False)
    return out
